# Initial kernel scaffold; baseline (speedup 1.0000x reference)
import jax
import jax.numpy as jnp
from jax.experimental import pallas as pl


def kernel(x, input):
    raise NotImplementedError("write your pallas kernel here")


# speedup vs baseline: 1.0088x; 1.0088x over previous
import jax
import jax.numpy as jnp
from jax.experimental import pallas as pl
from jax.experimental.pallas import tpu as pltpu

_LRATE = 0.01
_SIZE = 8192
_ROWS = 256

def _body(x_ref, inp_ref, out_ref, y_ref):
    @pl.when(pl.program_id(0) == 0)
    def _():
        _SIGN = jnp.int32(-2147483648)
        _LOW31 = jnp.int32(2147483647)
        x2 = x_ref[...]
        ib = jax.lax.bitcast_convert_type(x2, jnp.int32)
        key = jnp.where(ib >= 0, ib, ib ^ _LOW31)
        rank = jnp.int32((_SIZE - 1) // 2 + 1)

        def step(t, res_b):
            trial_b = res_b | (jnp.int32(1) << (31 - t))
            trial_s = trial_b ^ _SIGN
            cnt = jnp.sum((key < trial_s).astype(jnp.int32))
            return jnp.where(cnt >= rank, res_b, trial_b)

        res_b = jax.lax.fori_loop(0, 32, step, jnp.int32(0))
        med_s = res_b ^ _SIGN
        med_i = jnp.where(med_s >= 0, med_s, med_s ^ _LOW31)
        med_f = jax.lax.bitcast_convert_type(med_i, jnp.float32)
        y_ref[...] = jnp.where(x2 > med_f, jnp.float32(1.0), jnp.float32(0.0))

    a = inp_ref[...] * jnp.float32(-_LRATE)
    out_ref[...] = a * y_ref[...]


def kernel(x, input):
    x2 = x.reshape(1, _SIZE)
    inp2 = input.reshape(_SIZE, 1)
    return pl.pallas_call(
        _body,
        grid=(_SIZE // _ROWS,),
        in_specs=[
            pl.BlockSpec((1, _SIZE), lambda i: (0, 0)),
            pl.BlockSpec((_ROWS, 1), lambda i: (i, 0)),
        ],
        out_specs=pl.BlockSpec((_ROWS, _SIZE), lambda i: (i, 0)),
        out_shape=jax.ShapeDtypeStruct((_SIZE, _SIZE), jnp.float32),
        scratch_shapes=[pltpu.VMEM((1, _SIZE), jnp.float32)],
    )(x2, inp2)
